# Initial kernel scaffold; baseline (speedup 1.0000x reference)
import jax
import jax.numpy as jnp
from jax.experimental import pallas as pl


def kernel(x, edge_index, W1, b1, g1, be1, W2, b2, g2, be2, W3, b3):
    raise NotImplementedError("write your pallas kernel here")


# speedup vs baseline: 10.6766x; 10.6766x over previous
import functools

import jax
import jax.numpy as jnp
from jax import lax
from jax.experimental import pallas as pl
from jax.experimental.pallas import tpu as pltpu
from jax.experimental.pallas import tpu_sc as plsc

N = 10000
D_H = 128
D_OUT = 64
EPS = 1e-5

NC, NS, LANES = 2, 16, 16
NW = NC * NS
B = 128
N_PAD = 10112
ROWS_PER_TILE = N_PAD // NS
PAD_DST = N + 8


def _sc_mesh():
    return plsc.VectorSubcoreMesh(core_axis_name="c", subcore_axis_name="s")


@functools.lru_cache(maxsize=None)
def _deg_kernel(kb):
    @functools.partial(
        pl.kernel,
        out_type=jax.ShapeDtypeStruct((NC, N_PAD, D_H), jnp.float32),
        mesh=_sc_mesh(),
        scratch_types=[
            pltpu.VMEM((kb, B), jnp.int32),
            pltpu.VMEM((B, D_H), jnp.float32),
            pltpu.VMEM_SHARED((N_PAD, D_H), jnp.float32),
        ],
    )
    def deg_kernel(dst_hbm, ones_hbm, zeros_hbm, out_hbm, dst_v, ones_v, acc):
        c = lax.axis_index("c")
        s = lax.axis_index("s")
        wid = s * NC + c
        row0 = s * ROWS_PER_TILE
        pltpu.sync_copy(zeros_hbm.at[pl.ds(row0, ROWS_PER_TILE)],
                        acc.at[pl.ds(row0, ROWS_PER_TILE)])
        pltpu.sync_copy(dst_hbm.at[wid], dst_v)
        pltpu.sync_copy(ones_hbm, ones_v)
        plsc.subcore_barrier()

        def body(j, carry):
            pltpu.sync_copy(ones_v, acc.at[dst_v.at[j]], add=True)
            return carry

        lax.fori_loop(0, kb, body, 0)
        plsc.subcore_barrier()
        pltpu.sync_copy(acc.at[pl.ds(row0, ROWS_PER_TILE)],
                        out_hbm.at[c].at[pl.ds(row0, ROWS_PER_TILE)])

    return deg_kernel


@functools.lru_cache(maxsize=None)
def _feat_kernel(kb, d):
    @functools.partial(
        pl.kernel,
        out_type=jax.ShapeDtypeStruct((NC, N_PAD, d), jnp.float32),
        mesh=_sc_mesh(),
        scratch_types=[
            pltpu.VMEM((kb, B), jnp.int32),
            pltpu.VMEM((kb, B), jnp.int32),
            pltpu.VMEM((B, d), jnp.float32),
            pltpu.VMEM_SHARED((N_PAD, d), jnp.float32),
            pltpu.SemaphoreType.DMA,
        ],
    )
    def feat_kernel(y_hbm, src_hbm, dst_hbm, zeros_hbm, out_hbm,
                    src_v, dst_v, buf, acc, gsem):
        c = lax.axis_index("c")
        s = lax.axis_index("s")
        wid = s * NC + c
        row0 = s * ROWS_PER_TILE
        pltpu.sync_copy(zeros_hbm.at[pl.ds(row0, ROWS_PER_TILE)],
                        acc.at[pl.ds(row0, ROWS_PER_TILE)])
        pltpu.sync_copy(src_hbm.at[wid], src_v)
        pltpu.sync_copy(dst_hbm.at[wid], dst_v)
        plsc.subcore_barrier()

        def body(j, carry):
            pltpu.async_copy(y_hbm.at[src_v.at[j]], buf, gsem).wait()
            pltpu.sync_copy(buf, acc.at[dst_v.at[j]], add=True)
            return carry

        lax.fori_loop(0, kb, body, 0)
        plsc.subcore_barrier()
        pltpu.sync_copy(acc.at[pl.ds(row0, ROWS_PER_TILE)],
                        out_hbm.at[c].at[pl.ds(row0, ROWS_PER_TILE)])

    return feat_kernel


def _prep(x, w, degp):
    def body(x_ref, w_ref, degp_ref, y_ref, dis_ref):
        deg = 1.0 + degp_ref[0, :, 0:1] + degp_ref[1, :, 0:1]
        dis = lax.rsqrt(deg)
        dis_ref[...] = dis
        xw = jnp.dot(x_ref[...], w_ref[...], preferred_element_type=jnp.float32)
        y_ref[...] = xw * dis[:N]

    return pl.pallas_call(
        body,
        out_shape=(jax.ShapeDtypeStruct((N, w.shape[1]), jnp.float32),
                   jax.ShapeDtypeStruct((N_PAD, 1), jnp.float32)),
    )(x, w, degp)


def _combine_mid(z, y, dis, b, g, be, w_next):
    def body(z_ref, y_ref, dis_ref, b_ref, g_ref, be_ref, w_ref, o_ref):
        dis_n = dis_ref[:N]
        o = (z_ref[0, :N, :] + z_ref[1, :N, :] + y_ref[...]) * dis_n + b_ref[...]
        mean = jnp.mean(o, axis=0, keepdims=True)
        var = jnp.mean((o - mean) ** 2, axis=0, keepdims=True)
        h = g_ref[...] * (o - mean) * lax.rsqrt(var + EPS) + be_ref[...]
        h = jnp.maximum(h, 0.0)
        o_ref[...] = jnp.dot(h, w_ref[...], preferred_element_type=jnp.float32) * dis_n

    return pl.pallas_call(
        body,
        out_shape=jax.ShapeDtypeStruct((N, w_next.shape[1]), jnp.float32),
    )(z, y, dis, b.reshape(1, -1), g.reshape(1, -1), be.reshape(1, -1), w_next)


def _final(z, y, dis, b):
    d = b.shape[0]

    def body(z_ref, y_ref, dis_ref, b_ref, o_ref):
        o_ref[...] = ((z_ref[0, :N, :d] + z_ref[1, :N, :d] + y_ref[:, :d])
                      * dis_ref[:N] + b_ref[...])

    return pl.pallas_call(
        body,
        out_shape=jax.ShapeDtypeStruct((N, d), jnp.float32),
    )(z, y, dis, b.reshape(1, -1))


def kernel(x, edge_index, W1, b1, g1, be1, W2, b2, g2, be2, W3, b3):
    src, dst = edge_index[0], edge_index[1]
    e = src.shape[0]
    kb = -(-e // (NW * B))
    pad = NW * B * kb - e
    src_p = jnp.concatenate(
        [src, jnp.zeros((pad,), jnp.int32)]).reshape(NW, kb, B)
    dst_p = jnp.concatenate(
        [dst, jnp.full((pad,), PAD_DST, jnp.int32)]).reshape(NW, kb, B)
    ones128 = jnp.ones((B, D_H), jnp.float32)
    zeros128 = jnp.zeros((N_PAD, D_H), jnp.float32)

    w3p = jnp.pad(W3, ((0, 0), (0, D_H - D_OUT)))

    degp = _deg_kernel(kb)(dst_p, ones128, zeros128)
    y1, dis = _prep(x, W1, degp)
    z1 = _feat_kernel(kb, D_H)(y1, src_p, dst_p, zeros128)
    y2 = _combine_mid(z1, y1, dis, b1, g1, be1, W2)
    z2 = _feat_kernel(kb, D_H)(y2, src_p, dst_p, zeros128)
    y3 = _combine_mid(z2, y2, dis, b2, g2, be2, w3p)
    z3 = _feat_kernel(kb, D_H)(y3, src_p, dst_p, zeros128)
    return _final(z3, y3, dis, b3)
